# baseline (speedup 1.0000x reference)
import functools

import jax
import jax.numpy as jnp
from jax import lax
from jax.experimental import pallas as pl
from jax.experimental.pallas import tpu as pltpu
from jax.experimental.pallas import tpu_sc as plsc

NW = 32
NTILE = 16
B = 128
GROUP = 8


@functools.lru_cache(maxsize=None)
def _make_sc_agg(npad: int, width: int, cpw: int, with_deg: bool):
    rows_per_tile = npad // NTILE
    groups = cpw // GROUP
    mesh = plsc.VectorSubcoreMesh(core_axis_name="c", subcore_axis_name="s")

    out_type = [jax.ShapeDtypeStruct((2, npad, width), jnp.float32)]
    scratch = [
        pltpu.VMEM((GROUP, B), jnp.int32),
        pltpu.VMEM((GROUP, B), jnp.int32),
        pltpu.VMEM((B, width), jnp.float32),
        pltpu.VMEM((B, width), jnp.float32),
        pltpu.VMEM_SHARED((npad, width), jnp.float32),
        pltpu.SemaphoreType.DMA,
        pltpu.SemaphoreType.DMA,
    ]
    if with_deg:
        out_type.append(
            jax.ShapeDtypeStruct((NW * npad,), jnp.float32))
        scratch.append(pltpu.VMEM((npad,), jnp.float32))

    @functools.partial(
        pl.kernel, out_type=tuple(out_type), mesh=mesh,
        scratch_types=scratch,
        compiler_params=pltpu.CompilerParams(needs_layout_passes=False))
    def sc_agg(table_hbm, src_hbm, dst_hbm, out_hbm, *rest):
        if with_deg:
            (deg_hbm, src_g, dst_g, rows0, rows1, acc_sh, sem0, sem1,
             deg_v) = rest
        else:
            src_g, dst_g, rows0, rows1, acc_sh, sem0, sem1 = rest
        c = lax.axis_index("c")
        s = lax.axis_index("s")
        w = s * 2 + c

        zvec = jnp.zeros((16,), jnp.float32)

        def zrow(i, carry):
            for k in range(width // 16):
                rows0[i, pl.ds(k * 16, 16)] = zvec
                rows1[i, pl.ds(k * 16, 16)] = zvec
            return carry

        lax.fori_loop(0, B, zrow, 0)

        base = s * rows_per_tile
        nfull = rows_per_tile // B
        rem = rows_per_tile % B
        for t in range(nfull):
            pltpu.sync_copy(rows0, acc_sh.at[pl.ds(base + t * B, B)])
        if rem:
            pltpu.sync_copy(rows0.at[pl.ds(0, rem)],
                            acc_sh.at[pl.ds(base + nfull * B, rem)])

        if with_deg:
            def zdrow(i, carry):
                deg_v[pl.ds(i * 16, 16)] = zvec
                return carry

            lax.fori_loop(0, npad // 16, zdrow, 0)

        plsc.subcore_barrier()

        ones16 = jnp.ones((16,), jnp.float32)
        bufs = (rows0, rows1)
        sems = (sem0, sem1)

        def group_body(g, carry):
            gbase = (w * groups + g) * GROUP
            pltpu.sync_copy(src_hbm.at[pl.ds(gbase, GROUP)], src_g)
            pltpu.sync_copy(dst_hbm.at[pl.ds(gbase, GROUP)], dst_g)
            handles = [pltpu.async_copy(
                table_hbm.at[src_g.at[0]], bufs[0], sems[0])]
            for k in range(GROUP):
                if k + 1 < GROUP:
                    handles.append(pltpu.async_copy(
                        table_hbm.at[src_g.at[k + 1]],
                        bufs[(k + 1) % 2], sems[(k + 1) % 2]))
                if with_deg:
                    for t in range(B // 16):
                        dvec = dst_g[k, pl.ds(t * 16, 16)]
                        plsc.addupdate_scatter(deg_v, [dvec], ones16)
                handles[k].wait()
                pltpu.sync_copy(bufs[k % 2], acc_sh.at[dst_g.at[k]], add=True)
            return carry

        lax.fori_loop(0, groups, group_body, 0)

        plsc.subcore_barrier()

        pltpu.sync_copy(acc_sh.at[pl.ds(base, rows_per_tile)],
                        out_hbm.at[c, pl.ds(base, rows_per_tile)])
        if with_deg:
            pltpu.sync_copy(deg_v, deg_hbm.at[pl.ds(w * npad, npad)])

    return sc_agg


def _tc_dense_body(n_real, acc_ref, x_ref, inv_ref, u_ref,
                   w1_ref, b1_ref, w2_ref, b2_ref, out_ref):
    n = x_ref.shape[0]
    d = w1_ref.shape[0]
    rows = n // 128
    agg = acc_ref[0] + acc_ref[1] + x_ref[...]
    inv2d = inv_ref[...]
    inv3 = inv2d[:, :, None]
    hn = (agg.reshape(rows, 128, d) * inv3).reshape(n, d)
    z = (jnp.dot(hn, w1_ref[...], preferred_element_type=jnp.float32)
         + b1_ref[...])
    h1 = jnp.maximum(z, 0.0)
    nrm = jnp.sqrt(jnp.sum(h1 * h1, axis=1, keepdims=True))
    h = h1 / jnp.maximum(nrm, 1e-12)
    coef = jnp.sum(u_ref[...], axis=0) + inv2d
    h3 = h.reshape(rows, 128, d)
    t3 = h3 * (coef * jnp.where(inv2d > 0.0, 1.0, 0.0))[:, :, None]
    s1 = jnp.sum(t3, axis=0)
    m = jnp.sum(s1, axis=0, keepdims=True) * (1.0 / n_real)
    out_ref[...] = (
        jnp.dot(m, w2_ref[...], preferred_element_type=jnp.float32)
        + b2_ref[...])


@functools.lru_cache(maxsize=None)
def _make_sc_deg_u(npad: int, cpw: int, n_real: int):
    groups = cpw // GROUP
    chunks_pad = cpw * NW
    dchunks = chunks_pad // NTILE
    dgroups = dchunks // GROUP
    degrows = npad // 128
    mesh = plsc.VectorSubcoreMesh(core_axis_name="c", subcore_axis_name="s")

    @functools.partial(
        pl.kernel,
        out_type=(
            jax.ShapeDtypeStruct((degrows, 128), jnp.float32),
            jax.ShapeDtypeStruct((NW, degrows, 128), jnp.float32),
        ),
        mesh=mesh,
        scratch_types=[
            pltpu.VMEM((GROUP, B), jnp.int32),
            pltpu.VMEM((GROUP, B), jnp.int32),
            pltpu.VMEM((degrows, 128), jnp.float32),
            pltpu.VMEM((degrows, 128), jnp.float32),
            pltpu.VMEM((degrows,), jnp.int32),
            pltpu.VMEM_SHARED((degrows, 128), jnp.float32),
        ],
        compiler_params=pltpu.CompilerParams(needs_layout_passes=False))
    def sc_deg_u(src_hbm, dst_hbm, inv_hbm, u_hbm, src_g, dst_g,
                 deg_v, u_v, idx_id, deg_sh):
        c = lax.axis_index("c")
        s = lax.axis_index("s")
        w = s * 2 + c

        zvec = jnp.zeros((16,), jnp.float32)
        ones16 = jnp.ones((16,), jnp.float32)

        def zrow(i, carry):
            for k in range(8):
                deg_v[i, pl.ds(k * 16, 16)] = zvec
                u_v[i, pl.ds(k * 16, 16)] = zvec
            return carry

        lax.fori_loop(0, degrows, zrow, 0)
        for t in range(degrows // 16):
            idx_id[pl.ds(t * 16, 16)] = lax.iota(jnp.int32, 16) + t * 16

        @pl.when(s == 0)
        def _():
            pltpu.sync_copy(u_v, deg_sh)

        plsc.subcore_barrier()

        def deg_group(g, carry):
            gbase = (s * dgroups + g) * GROUP
            pltpu.sync_copy(dst_hbm.at[pl.ds(gbase, GROUP)], dst_g)
            for k in range(GROUP):
                for t in range(B // 16):
                    dvec = dst_g[k, pl.ds(t * 16, 16)]
                    plsc.addupdate_scatter(
                        deg_v, [dvec >> 7, dvec & 127], ones16)
            return carry

        lax.fori_loop(0, dgroups, deg_group, 0)

        pltpu.sync_copy(deg_v, deg_sh.at[idx_id], add=True)
        plsc.subcore_barrier()
        pltpu.sync_copy(deg_sh, deg_v)

        def inv_row(i, carry):
            for k in range(8):
                vid = i * 128 + k * 16 + lax.iota(jnp.int32, 16)
                dv = deg_v[i, pl.ds(k * 16, 16)]
                deg_v[i, pl.ds(k * 16, 16)] = jnp.where(
                    vid < n_real, 1.0 / (dv + 1.0), 0.0)
            return carry

        lax.fori_loop(0, degrows, inv_row, 0)

        @pl.when(jnp.logical_and(c == 0, s == 0))
        def _():
            pltpu.sync_copy(deg_v, inv_hbm)

        def u_group(g, carry):
            gbase = (w * groups + g) * GROUP
            pltpu.sync_copy(src_hbm.at[pl.ds(gbase, GROUP)], src_g)
            pltpu.sync_copy(dst_hbm.at[pl.ds(gbase, GROUP)], dst_g)
            for k in range(GROUP):
                for t in range(B // 16):
                    svec = src_g[k, pl.ds(t * 16, 16)]
                    dvec = dst_g[k, pl.ds(t * 16, 16)]
                    w16 = plsc.load_gather(deg_v, [dvec >> 7, dvec & 127])
                    plsc.addupdate_scatter(
                        u_v, [svec >> 7, svec & 127], w16)
            return carry

        lax.fori_loop(0, groups, u_group, 0)

        pltpu.sync_copy(u_v, u_hbm.at[w])

    return sc_deg_u


def kernel(x, edge_index, W1, b1, W2, b2):
    n, d = x.shape
    e = edge_index.shape[1]
    npad = -(-(n + 1) // 2048) * 2048
    chunks = -(-e // B)
    cpw = -(-chunks // (NW * 8)) * 8
    chunks_pad = cpw * NW
    epad = chunks_pad * B

    src = edge_index[0]
    dst = edge_index[1]
    pad = epad - e
    fill = n + jnp.arange(pad, dtype=jnp.int32) % (npad - n)
    src_p = (jnp.concatenate([src, fill]).reshape(cpw, NW, B)
             .transpose(1, 0, 2).reshape(chunks_pad, B))
    dst_p = (jnp.concatenate([dst, fill]).reshape(cpw, NW, B)
             .transpose(1, 0, 2).reshape(chunks_pad, B))

    x_pad = jnp.concatenate([x, jnp.zeros((npad - n, d), jnp.float32)], axis=0)

    inv2d, u_part = _make_sc_deg_u(npad, cpw, n)(src_p, dst_p)
    (acc1,) = _make_sc_agg(npad, d, cpw, False)(x_pad, src_p, dst_p)

    out = pl.pallas_call(
        functools.partial(_tc_dense_body, float(n)),
        out_shape=jax.ShapeDtypeStruct((1, d), jnp.float32),
    )(acc1, x_pad, inv2d, u_part, W1, b1.reshape(1, d), W2,
      b2.reshape(1, d))

    return out
